# baseline (speedup 1.0000x reference)
import functools

import jax
import jax.numpy as jnp
from jax import lax
from jax.experimental import pallas as pl
from jax.experimental.pallas import tpu as pltpu
from jax.experimental.pallas import tpu_sc as plsc

B = 16384
D = 32
V = 100001

_info = plsc.get_sparse_core_info()
_NC = _info.num_cores
_NS = _info.num_subcores
_NW = _NC * _NS
_BPW = B // _NW

_mesh = plsc.VectorSubcoreMesh(core_axis_name="c", subcore_axis_name="s")


@functools.partial(
    pl.kernel,
    mesh=_mesh,
    out_type=jax.ShapeDtypeStruct((B, D), jnp.float32),
    scratch_types=[
        pltpu.VMEM((_BPW,), jnp.int32),
        pltpu.VMEM((_BPW, D), jnp.float32),
        pltpu.SemaphoreType.DMA,
    ],
    compiler_params=pltpu.CompilerParams(use_tc_tiling_on_sc=False),
)
def _sc_gather(table_hbm, idx_hbm, out_hbm, idx_v, rows_v, sem):
    wid = lax.axis_index("s") * _NC + lax.axis_index("c")
    base = wid * _BPW
    pltpu.sync_copy(idx_hbm.at[pl.ds(base, _BPW)], idx_v)
    pltpu.async_copy(table_hbm.at[idx_v], rows_v, sem).wait()
    pltpu.sync_copy(rows_v, out_hbm.at[pl.ds(base, _BPW)])


def _mlp_body(x_ref, w1_ref, b1_ref, w2_ref, b2_ref, o_ref):
    x = x_ref[...]
    h = jnp.maximum(
        jnp.dot(x, w1_ref[...], preferred_element_type=jnp.float32)
        + b1_ref[...],
        0.0,
    )
    o_ref[...] = (
        jnp.dot(h, w2_ref[...], preferred_element_type=jnp.float32)
        + b2_ref[...]
    )


def _packed_mlp(x_packed, W1p, b1p, W2p, b2p):
    blk = 1024
    n = x_packed.shape[0]
    return pl.pallas_call(
        _mlp_body,
        grid=(n // blk,),
        in_specs=[
            pl.BlockSpec((blk, 4 * D), lambda i: (i, 0)),
            pl.BlockSpec(W1p.shape, lambda i: (0, 0)),
            pl.BlockSpec((1, W1p.shape[1]), lambda i: (0, 0)),
            pl.BlockSpec(W2p.shape, lambda i: (0, 0)),
            pl.BlockSpec((1, W2p.shape[1]), lambda i: (0, 0)),
        ],
        out_specs=pl.BlockSpec((blk, 4 * D), lambda i: (i, 0)),
        out_shape=jax.ShapeDtypeStruct((n, 4 * D), jnp.float32),
    )(x_packed, W1p, b1p.reshape(1, -1), W2p, b2p.reshape(1, -1))


def kernel(user_id, table, W1, b1, W2, b2):
    uid = user_id.astype(jnp.int32)
    one = lax.optimization_barrier(jnp.float32(1.0))
    t_lin = table * one
    gathered = _sc_gather(t_lin, uid)

    eye4 = jnp.eye(4, dtype=jnp.float32)
    W1p = jnp.kron(eye4, W1)
    W2p = jnp.kron(eye4, W2)
    b1p = jnp.tile(b1, 4)
    b2p = jnp.tile(b2, 4)

    x_packed = gathered.reshape(B // 4, 4 * D)
    out_packed = _packed_mlp(x_packed, W1p, b1p, W2p, b2p)
    return out_packed.reshape(B, D) * one


# speedup vs baseline: 1.4161x; 1.4161x over previous
import functools

import jax
import jax.numpy as jnp
from jax import lax
from jax.experimental import pallas as pl
from jax.experimental.pallas import tpu as pltpu
from jax.experimental.pallas import tpu_sc as plsc

B = 16384
D = 32
V = 100001
LBLK = 2048
NQB = 13
Q = NQB * LBLK
NVBLK = -(-V // LBLK)

_info = plsc.get_sparse_core_info()
_NC = _info.num_cores
_NS = _info.num_subcores
_NW = _NC * _NS
_BPW = B // _NW

_mesh = plsc.VectorSubcoreMesh(core_axis_name="c", subcore_axis_name="s")


def _t1_body(x0, x1, x2, x3, w1t_ref, b1_ref, w2t_ref, b2_ref, o_ref):
    w1t = w1t_ref[...]
    b1 = b1_ref[...]
    w2t = w2t_ref[...]
    b2 = b2_ref[...]
    cols = []
    for xc in (x0, x1, x2, x3):
        x = xc[...]
        h = jnp.maximum(
            jnp.dot(w1t, x, preferred_element_type=jnp.float32) + b1, 0.0
        )
        ot = (
            jnp.dot(w2t, h, preferred_element_type=jnp.float32) + b2
        )
        cols.append(ot.T)
    o_ref[...] = jnp.concatenate(cols, axis=1)


def _t1(tableT, W1T, b1, W2T, b2):
    def tmap(c):
        return lambda i: (0, jnp.minimum(NQB * c + i, NVBLK - 1))

    return pl.pallas_call(
        _t1_body,
        grid=(NQB,),
        in_specs=[
            pl.BlockSpec((D, LBLK), tmap(0)),
            pl.BlockSpec((D, LBLK), tmap(1)),
            pl.BlockSpec((D, LBLK), tmap(2)),
            pl.BlockSpec((D, LBLK), tmap(3)),
            pl.BlockSpec(W1T.shape, lambda i: (0, 0)),
            pl.BlockSpec((W1T.shape[0], 1), lambda i: (0, 0)),
            pl.BlockSpec(W2T.shape, lambda i: (0, 0)),
            pl.BlockSpec((W2T.shape[0], 1), lambda i: (0, 0)),
        ],
        out_specs=pl.BlockSpec((LBLK, 4 * D), lambda i: (i, 0)),
        out_shape=jax.ShapeDtypeStruct((Q, 4 * D), jnp.float32),
    )(
        tableT, tableT, tableT, tableT,
        W1T, b1.reshape(-1, 1), W2T, b2.reshape(-1, 1),
    )


@functools.partial(
    pl.kernel,
    mesh=_mesh,
    out_type=jax.ShapeDtypeStruct((B, 4 * D), jnp.float32),
    scratch_types=[
        pltpu.VMEM((_BPW,), jnp.int32),
        pltpu.VMEM((_BPW,), jnp.int32),
        pltpu.VMEM((_BPW, 4 * D), jnp.float32),
        pltpu.SemaphoreType.DMA,
    ],
)
def _sc_gather(lines_hbm, idx_hbm, out_hbm, idx_v, j_v, rows_v, sem):
    wid = lax.axis_index("s") * _NC + lax.axis_index("c")
    base = wid * _BPW
    pltpu.sync_copy(idx_hbm.at[pl.ds(base, _BPW)], idx_v)
    for k in range(_BPW // 16):
        sl = pl.ds(k * 16, 16)
        j_v[sl] = lax.rem(idx_v[sl], Q)
    pltpu.async_copy(lines_hbm.at[j_v], rows_v, sem).wait()
    pltpu.sync_copy(rows_v, out_hbm.at[pl.ds(base, _BPW)])


def _t3_body(g_ref, uid_ref, o_ref):
    uid = uid_ref[...]
    slot = (
        (uid >= Q).astype(jnp.int32)
        + (uid >= 2 * Q).astype(jnp.int32)
        + (uid >= 3 * Q).astype(jnp.int32)
    )
    g = g_ref[...]
    x = jnp.where(
        slot < 2,
        jnp.where(slot == 0, g[:, 0:D], g[:, D:2 * D]),
        jnp.where(slot == 2, g[:, 2 * D:3 * D], g[:, 3 * D:4 * D]),
    )
    o_ref[...] = x.T


def _t3(gathered, user_id):
    blk = 4096
    return pl.pallas_call(
        _t3_body,
        grid=(B // blk,),
        in_specs=[
            pl.BlockSpec((blk, 4 * D), lambda i: (i, 0)),
            pl.BlockSpec((blk, 1), lambda i: (i, 0)),
        ],
        out_specs=pl.BlockSpec((D, blk), lambda i: (0, i)),
        out_shape=jax.ShapeDtypeStruct((D, B), jnp.float32),
    )(gathered, user_id.reshape(B, 1))


def kernel(user_id, table, W1, b1, W2, b2):
    uid = user_id.astype(jnp.int32)
    lines = _t1(table.T, W1.T, b1, W2.T, b2)
    gathered = _sc_gather(lines, uid)
    return _t3(gathered, uid).T


# speedup vs baseline: 1.6293x; 1.1505x over previous
import functools

import jax
import jax.numpy as jnp
from jax import lax
from jax.experimental import pallas as pl
from jax.experimental.pallas import tpu as pltpu
from jax.experimental.pallas import tpu_sc as plsc

B = 16384
D = 32
V = 100001
LBLK = 2048
NQB = 13
Q = NQB * LBLK
NVBLK = -(-V // LBLK)

_info = plsc.get_sparse_core_info()
_NC = _info.num_cores
_NS = _info.num_subcores
_NW = _NC * _NS
_BPW = B // _NW

_mesh = plsc.VectorSubcoreMesh(core_axis_name="c", subcore_axis_name="s")

_DN_T = (((0,), (0,)), ((), ()))
_DN = (((1,), (0,)), ((), ()))


def _t1_body(x0, x1, x2, x3, w1_ref, b1_ref, w2_ref, b2_ref, o_ref):
    w1 = w1_ref[...]
    b1 = b1_ref[...]
    w2 = w2_ref[...]
    b2 = b2_ref[...]
    cols = []
    for xc in (x0, x1, x2, x3):
        x = xc[...]
        h = jnp.maximum(
            lax.dot_general(x, w1, _DN_T, preferred_element_type=jnp.float32)
            + b1,
            0.0,
        )
        cols.append(
            lax.dot_general(h, w2, _DN, preferred_element_type=jnp.float32)
            + b2
        )
    o_ref[...] = jnp.concatenate(cols, axis=1)


def _t1(tableT, W1, b1, W2, b2):
    def tmap(c):
        return lambda i: (0, jnp.minimum(NQB * c + i, NVBLK - 1))

    return pl.pallas_call(
        _t1_body,
        grid=(NQB,),
        in_specs=[
            pl.BlockSpec((D, LBLK), tmap(0)),
            pl.BlockSpec((D, LBLK), tmap(1)),
            pl.BlockSpec((D, LBLK), tmap(2)),
            pl.BlockSpec((D, LBLK), tmap(3)),
            pl.BlockSpec(W1.shape, lambda i: (0, 0)),
            pl.BlockSpec((1, W1.shape[1]), lambda i: (0, 0)),
            pl.BlockSpec(W2.shape, lambda i: (0, 0)),
            pl.BlockSpec((1, W2.shape[1]), lambda i: (0, 0)),
        ],
        out_specs=pl.BlockSpec((LBLK, 4 * D), lambda i: (i, 0)),
        out_shape=jax.ShapeDtypeStruct((Q, 4 * D), jnp.float32),
    )(
        tableT, tableT, tableT, tableT,
        W1, b1.reshape(1, -1), W2, b2.reshape(1, -1),
    )


@functools.partial(
    pl.kernel,
    mesh=_mesh,
    out_type=jax.ShapeDtypeStruct((D, B), jnp.float32),
    scratch_types=[
        pltpu.VMEM((_BPW,), jnp.int32),
        pltpu.VMEM((_BPW,), jnp.int32),
        pltpu.VMEM((_BPW,), jnp.int32),
        pltpu.VMEM((_BPW, 4 * D), jnp.float32),
        pltpu.VMEM((D, _BPW), jnp.float32),
        pltpu.SemaphoreType.DMA,
    ],
    compiler_params=pltpu.CompilerParams(needs_layout_passes=False),
)
def _sc_gather(lines_hbm, idx_hbm, out_hbm, idx_v, j_v, col_v, rows_v,
               outT_v, sem):
    wid = lax.axis_index("s") * _NC + lax.axis_index("c")
    base = wid * _BPW
    pltpu.sync_copy(idx_hbm.at[pl.ds(base, _BPW)], idx_v)
    for k in range(_BPW // 16):
        sl = pl.ds(k * 16, 16)
        v = idx_v[sl]
        slot = lax.div(v, Q)
        j_v[sl] = v - slot * Q
        col_v[sl] = slot * D
    pltpu.async_copy(lines_hbm.at[j_v], rows_v, sem).wait()

    row_iota = lax.iota(jnp.int32, 16)
    for k in range(_BPW // 16):
        rsl = pl.ds(k * 16, 16)
        rows16 = row_iota + k * 16
        col0 = col_v[rsl]
        for d in range(D):
            vals = plsc.load_gather(rows_v, [rows16, col0 + d])
            outT_v[d, rsl] = vals
    pltpu.sync_copy(outT_v, out_hbm.at[:, pl.ds(base, _BPW)])


def kernel(user_id, table, W1, b1, W2, b2):
    uid = user_id.astype(jnp.int32)
    lines = _t1(table.T, W1, b1, W2, b2)
    return _sc_gather(lines, uid).T


# speedup vs baseline: 1.6356x; 1.0039x over previous
import functools

import jax
import jax.numpy as jnp
from jax import lax
from jax.experimental import pallas as pl
from jax.experimental.pallas import tpu as pltpu
from jax.experimental.pallas import tpu_sc as plsc

B = 16384
D = 32
V = 100001
LBLK = 2048
NQB = 13
Q = NQB * LBLK
NVBLK = -(-V // LBLK)

_info = plsc.get_sparse_core_info()
_NC = _info.num_cores
_NS = _info.num_subcores
_NW = _NC * _NS
_BPW = B // _NW

_mesh = plsc.VectorSubcoreMesh(core_axis_name="c", subcore_axis_name="s")

_DN_T = (((0,), (0,)), ((), ()))
_DN = (((1,), (0,)), ((), ()))


def _t1_body(x0, x1, x2, x3, w1_ref, b1_ref, w2_ref, b2_ref, o_ref):
    w1 = w1_ref[...]
    b1 = b1_ref[...]
    w2 = w2_ref[...]
    b2 = b2_ref[...]
    x = jnp.concatenate(
        [x0[...], x1[...], x2[...], x3[...]], axis=1
    )
    h = jnp.maximum(
        lax.dot_general(x, w1, _DN_T, preferred_element_type=jnp.float32)
        + b1,
        0.0,
    )
    ot = (
        lax.dot_general(h, w2, _DN, preferred_element_type=jnp.float32)
        + b2
    )
    o_ref[...] = jnp.concatenate(
        [ot[c * LBLK:(c + 1) * LBLK] for c in range(4)], axis=1
    )


def _t1(tableT, W1, b1, W2, b2):
    def tmap(c):
        return lambda i: (0, jnp.minimum(NQB * c + i, NVBLK - 1))

    return pl.pallas_call(
        _t1_body,
        grid=(NQB,),
        in_specs=[
            pl.BlockSpec((D, LBLK), tmap(0)),
            pl.BlockSpec((D, LBLK), tmap(1)),
            pl.BlockSpec((D, LBLK), tmap(2)),
            pl.BlockSpec((D, LBLK), tmap(3)),
            pl.BlockSpec(W1.shape, lambda i: (0, 0)),
            pl.BlockSpec((1, W1.shape[1]), lambda i: (0, 0)),
            pl.BlockSpec(W2.shape, lambda i: (0, 0)),
            pl.BlockSpec((1, W2.shape[1]), lambda i: (0, 0)),
        ],
        out_specs=pl.BlockSpec((LBLK, 4 * D), lambda i: (i, 0)),
        out_shape=jax.ShapeDtypeStruct((Q, 4 * D), jnp.float32),
    )(
        tableT, tableT, tableT, tableT,
        W1, b1.reshape(1, -1), W2, b2.reshape(1, -1),
    )


@functools.partial(
    pl.kernel,
    mesh=_mesh,
    out_type=jax.ShapeDtypeStruct((D, B), jnp.float32),
    scratch_types=[
        pltpu.VMEM((_BPW,), jnp.int32),
        pltpu.VMEM((_BPW,), jnp.int32),
        pltpu.VMEM((_BPW,), jnp.int32),
        pltpu.VMEM((_BPW, 4 * D), jnp.float32),
        pltpu.VMEM((D, _BPW), jnp.float32),
        pltpu.SemaphoreType.DMA,
    ],
    compiler_params=pltpu.CompilerParams(needs_layout_passes=False),
)
def _sc_gather(lines_hbm, idx_hbm, out_hbm, idx_v, j_v, col_v, rows_v,
               outT_v, sem):
    wid = lax.axis_index("s") * _NC + lax.axis_index("c")
    base = wid * _BPW
    pltpu.sync_copy(idx_hbm.at[pl.ds(base, _BPW)], idx_v)
    for k in range(_BPW // 16):
        sl = pl.ds(k * 16, 16)
        v = idx_v[sl]
        slot = lax.div(v, Q)
        j_v[sl] = v - slot * Q
        col_v[sl] = slot * D
    pltpu.async_copy(lines_hbm.at[j_v], rows_v, sem).wait()

    row_iota = lax.iota(jnp.int32, 16)
    for k in range(_BPW // 16):
        rsl = pl.ds(k * 16, 16)
        rows16 = row_iota + k * 16
        col0 = col_v[rsl]
        for d in range(D):
            vals = plsc.load_gather(rows_v, [rows16, col0 + d])
            outT_v[d, rsl] = vals
    pltpu.sync_copy(outT_v, out_hbm.at[:, pl.ds(base, _BPW)])


def kernel(user_id, table, W1, b1, W2, b2):
    uid = user_id.astype(jnp.int32)
    lines = _t1(table.T, W1, b1, W2, b2)
    return _sc_gather(lines, uid).T
